# Initial kernel scaffold; baseline (speedup 1.0000x reference)
import jax
import jax.numpy as jnp
from jax.experimental import pallas as pl


def kernel(x, pos_table):
    raise NotImplementedError("write your pallas kernel here")


# speedup vs baseline: 22.7572x; 22.7572x over previous
import jax
import jax.numpy as jnp
from jax.experimental import pallas as pl

_SEQ_LEN = 200
_BLOCK_B = 512


def _bcast_kernel(tab_ref, out_ref):
    out_ref[...] = jnp.broadcast_to(tab_ref[...], out_ref.shape)


def kernel(x, pos_table):
    batch = x.shape[0]
    seq_len = x.shape[1]
    emb = pos_table.shape[1]
    flat = seq_len * emb
    tab = pos_table[:seq_len].reshape(1, flat)

    block_b = _BLOCK_B if batch % _BLOCK_B == 0 else batch
    grid = (batch // block_b,)

    out = pl.pallas_call(
        _bcast_kernel,
        grid=grid,
        in_specs=[pl.BlockSpec((1, flat), lambda i: (0, 0))],
        out_specs=pl.BlockSpec((block_b, flat), lambda i: (i, 0)),
        out_shape=jax.ShapeDtypeStruct((batch, flat), pos_table.dtype),
    )(tab)
    return out.reshape(batch, seq_len, emb)
